# baseline (speedup 1.0000x reference)
import functools

import jax
import jax.numpy as jnp
from jax import lax
from jax.experimental import pallas as pl
from jax.experimental.pallas import tpu as pltpu
from jax.experimental.pallas import tpu_sc as plsc

_LANES = 16
_NUM_WORKERS = 32
_ICHUNK = 128
_L = 50
_D = 12
_CH = 6


def _card_embed_body(ranks_hbm, suits_hbm, rtab_hbm, stab_hbm, out_hbm,
                     rtab_v, stab_v, ctab_v, rk0, rk1, st0, st1, oa, ob,
                     sin0, sin1, sa, sb, n_i):
    num_cores = jax.lax.axis_size("c")
    wid = lax.axis_index("s") * num_cores + lax.axis_index("c")

    pltpu.sync_copy(rtab_hbm, rtab_v)
    pltpu.sync_copy(stab_hbm, stab_v)

    lanes = lax.iota(jnp.int32, _LANES)

    for k in range(5):
        t = jnp.minimum(lanes + _LANES * k, 69)
        r = t // 5
        s = t - r * 5
        for c in range(_D):
            cv = jnp.full((_LANES,), c if c < 8 else c - 8, jnp.int32)
            if c < 8:
                vals = plsc.load_gather(rtab_v, [r, cv])
            else:
                vals = plsc.load_gather(stab_v, [s, cv])
            plsc.store_scatter(ctab_v, [t * _D + c], vals)

    per_w = n_i // _NUM_WORKERS
    groups_per_j = _ICHUNK // _LANES
    nchunks = per_w // _ICHUNK

    def start_in(kc, rk, st, sem):
        i0 = wid * per_w + kc * _ICHUNK
        return (
            pltpu.async_copy(ranks_hbm.at[:, pl.ds(i0, _ICHUNK)], rk, sem),
            pltpu.async_copy(suits_hbm.at[:, pl.ds(i0, _ICHUNK)], st, sem),
        )

    def half(kc, c0, obuf, sem, rk, st):
        i0 = wid * per_w + kc * _ICHUNK

        @plsc.parallel_loop(0, _L * groups_per_j, step=1, unroll=4)
        def grp(g2):
            j = lax.shift_right_logical(g2, 3)
            g = lax.bitwise_and(g2, 7)
            col = g * _LANES
            r16 = rk[j, pl.ds(col, _LANES)]
            s16 = st[j, pl.ds(col, _LANES)]
            cidx = r16 * (5 * _D) + s16 * _D
            for cc in range(_CH):
                vals = plsc.load_gather(ctab_v, [cidx + (c0 + cc)])
                obuf[cc * _L + j, pl.ds(col, _LANES)] = vals

        return [
            pltpu.async_copy(obuf.at[pl.ds(cc * _L, _L), :],
                             out_hbm.at[c0 + cc, :, pl.ds(i0, _ICHUNK)],
                             sem)
            for cc in range(_CH)
        ]

    in_bufs = ((rk0, st0, sin0), (rk1, st1, sin1))
    in_handles = {0: start_in(0, *in_bufs[0])}
    pend = {"a": None, "b": None}

    for kc in range(nchunks):
        rk, st, _ = in_bufs[kc % 2]
        for h in in_handles.pop(kc):
            h.wait()
        if kc + 1 < nchunks:
            in_handles[kc + 1] = start_in(kc + 1, *in_bufs[(kc + 1) % 2])
        for key, c0, obuf, sem in (("a", 0, oa, sa), ("b", _CH, ob, sb)):
            if pend[key] is not None:
                for h in pend[key]:
                    h.wait()
            pend[key] = half(kc, c0, obuf, sem, rk, st)

    for key in ("a", "b"):
        for h in pend[key]:
            h.wait()


@jax.jit
def kernel(ranks, suits, rank_table, suit_table):
    B, L = ranks.shape
    per_w = B // _NUM_WORKERS
    assert per_w * _NUM_WORKERS == B and per_w % _ICHUNK == 0 and L == _L

    mesh = plsc.VectorSubcoreMesh(core_axis_name="c", subcore_axis_name="s")
    res = pl.kernel(
        functools.partial(_card_embed_body, n_i=B),
        out_type=jax.ShapeDtypeStruct((_D, _L, B), jnp.float32),
        mesh=mesh,
        compiler_params=pltpu.CompilerParams(needs_layout_passes=False),
        scratch_types=[
            pltpu.VMEM((14, 8), jnp.float32),
            pltpu.VMEM((5, 4), jnp.float32),
            pltpu.VMEM((70 * _D,), jnp.float32),
            pltpu.VMEM((_L, _ICHUNK), jnp.int32),
            pltpu.VMEM((_L, _ICHUNK), jnp.int32),
            pltpu.VMEM((_L, _ICHUNK), jnp.int32),
            pltpu.VMEM((_L, _ICHUNK), jnp.int32),
            pltpu.VMEM((_CH * _L, _ICHUNK), jnp.float32),
            pltpu.VMEM((_CH * _L, _ICHUNK), jnp.float32),
            pltpu.SemaphoreType.DMA,
            pltpu.SemaphoreType.DMA,
            pltpu.SemaphoreType.DMA,
            pltpu.SemaphoreType.DMA,
        ],
    )(ranks.T, suits.T, rank_table, suit_table)
    return res.transpose(2, 1, 0)


# speedup vs baseline: 1.0150x; 1.0150x over previous
import functools

import jax
import jax.numpy as jnp
from jax import lax
from jax.experimental import pallas as pl
from jax.experimental.pallas import tpu as pltpu
from jax.experimental.pallas import tpu_sc as plsc

_LANES = 16
_NUM_WORKERS = 32
_ICHUNK = 128
_L = 50
_D = 12
_CH = 6


def _card_embed_body(ranks_hbm, suits_hbm, rtab_hbm, stab_hbm, out_hbm,
                     rtab_v, stab_v, ctab_v, rk0, rk1, st0, st1, oa, ob,
                     sin0, sin1, sa, sb, n_i):
    num_cores = jax.lax.axis_size("c")
    wid = lax.axis_index("s") * num_cores + lax.axis_index("c")

    pltpu.sync_copy(rtab_hbm, rtab_v)
    pltpu.sync_copy(stab_hbm, stab_v)

    lanes = lax.iota(jnp.int32, _LANES)

    for k in range(5):
        t = jnp.minimum(lanes + _LANES * k, 69)
        r = t // 5
        s = t - r * 5
        for c in range(_D):
            cv = jnp.full((_LANES,), c if c < 8 else c - 8, jnp.int32)
            if c < 8:
                vals = plsc.load_gather(rtab_v, [cv, r])
            else:
                vals = plsc.load_gather(stab_v, [cv, s])
            plsc.store_scatter(ctab_v, [t * _D + c], vals)

    per_w = n_i // _NUM_WORKERS
    groups_per_j = _ICHUNK // _LANES
    nchunks = per_w // _ICHUNK

    def start_in(kc, rk, st, sem):
        i0 = wid * per_w + kc * _ICHUNK
        return (
            pltpu.async_copy(ranks_hbm.at[:, pl.ds(i0, _ICHUNK)], rk, sem),
            pltpu.async_copy(suits_hbm.at[:, pl.ds(i0, _ICHUNK)], st, sem),
        )

    def half(kc, c0, obuf, sem, rk, st):
        i0 = wid * per_w + kc * _ICHUNK

        @plsc.parallel_loop(0, _L * groups_per_j, step=1, unroll=4)
        def grp(g2):
            j = lax.shift_right_logical(g2, 3)
            g = lax.bitwise_and(g2, 7)
            col = g * _LANES
            r16 = rk[j, pl.ds(col, _LANES)]
            s16 = st[j, pl.ds(col, _LANES)]
            cidx = r16 * (5 * _D) + s16 * _D
            for cc in range(_CH):
                vals = plsc.load_gather(ctab_v, [cidx + (c0 + cc)])
                obuf[cc * _L + j, pl.ds(col, _LANES)] = vals

        return [
            pltpu.async_copy(obuf.at[pl.ds(cc * _L, _L), :],
                             out_hbm.at[c0 + cc, :, pl.ds(i0, _ICHUNK)],
                             sem)
            for cc in range(_CH)
        ]

    in_bufs = ((rk0, st0, sin0), (rk1, st1, sin1))
    in_handles = {0: start_in(0, *in_bufs[0])}
    pend = {"a": None, "b": None}

    for kc in range(nchunks):
        rk, st, _ = in_bufs[kc % 2]
        for h in in_handles.pop(kc):
            h.wait()
        if kc + 1 < nchunks:
            in_handles[kc + 1] = start_in(kc + 1, *in_bufs[(kc + 1) % 2])
        for key, c0, obuf, sem in (("a", 0, oa, sa), ("b", _CH, ob, sb)):
            if pend[key] is not None:
                for h in pend[key]:
                    h.wait()
            pend[key] = half(kc, c0, obuf, sem, rk, st)

    for key in ("a", "b"):
        for h in pend[key]:
            h.wait()


@jax.jit
def kernel(ranks, suits, rank_table, suit_table):
    B, L = ranks.shape
    per_w = B // _NUM_WORKERS
    assert per_w * _NUM_WORKERS == B and per_w % _ICHUNK == 0 and L == _L

    mesh = plsc.VectorSubcoreMesh(core_axis_name="c", subcore_axis_name="s")
    res = pl.kernel(
        functools.partial(_card_embed_body, n_i=B),
        out_type=jax.ShapeDtypeStruct((_D, _L, B), jnp.float32),
        mesh=mesh,
        compiler_params=pltpu.CompilerParams(needs_layout_passes=False),
        scratch_types=[
            pltpu.VMEM((8, 14), jnp.float32),
            pltpu.VMEM((4, 5), jnp.float32),
            pltpu.VMEM((70 * _D,), jnp.float32),
            pltpu.VMEM((_L, _ICHUNK), jnp.int32),
            pltpu.VMEM((_L, _ICHUNK), jnp.int32),
            pltpu.VMEM((_L, _ICHUNK), jnp.int32),
            pltpu.VMEM((_L, _ICHUNK), jnp.int32),
            pltpu.VMEM((_CH * _L, _ICHUNK), jnp.float32),
            pltpu.VMEM((_CH * _L, _ICHUNK), jnp.float32),
            pltpu.SemaphoreType.DMA,
            pltpu.SemaphoreType.DMA,
            pltpu.SemaphoreType.DMA,
            pltpu.SemaphoreType.DMA,
        ],
    )(ranks.T, suits.T, rank_table.T, suit_table.T)
    return res.transpose(2, 1, 0)


# speedup vs baseline: 1.0221x; 1.0071x over previous
import functools

import jax
import jax.numpy as jnp
from jax import lax
from jax.experimental import pallas as pl
from jax.experimental.pallas import tpu as pltpu
from jax.experimental.pallas import tpu_sc as plsc

_LANES = 16
_NUM_WORKERS = 32
_ICHUNK = 128
_L = 50
_D = 12
_CH = 6


def _card_embed_body(ranks_hbm, suits_hbm, rtab_hbm, stab_hbm, out_hbm,
                     rtab_v, stab_v, ctab_v, rk0, rk1, st0, st1, oa, ob,
                     sin0, sin1, sa, sb, n_i):
    num_cores = jax.lax.axis_size("c")
    wid = lax.axis_index("s") * num_cores + lax.axis_index("c")

    pltpu.sync_copy(rtab_hbm, rtab_v)
    pltpu.sync_copy(stab_hbm, stab_v)

    lanes = lax.iota(jnp.int32, _LANES)

    for k in range(5):
        t = jnp.minimum(lanes + _LANES * k, 69)
        r = t // 5
        s = t - r * 5
        for c in range(_D):
            if c < 8:
                vals = plsc.load_gather(rtab_v, [r * 8 + c])
            else:
                vals = plsc.load_gather(stab_v, [s * 4 + (c - 8)])
            plsc.store_scatter(ctab_v, [t * _D + c], vals)

    per_w = n_i // _NUM_WORKERS
    groups_per_j = _ICHUNK // _LANES
    nchunks = per_w // _ICHUNK

    def start_in(kc, rk, st, sem):
        i0 = wid * per_w + kc * _ICHUNK
        return (
            pltpu.async_copy(ranks_hbm.at[:, pl.ds(i0, _ICHUNK)], rk, sem),
            pltpu.async_copy(suits_hbm.at[:, pl.ds(i0, _ICHUNK)], st, sem),
        )

    def half(kc, c0, obuf, sem, rk, st):
        i0 = wid * per_w + kc * _ICHUNK

        @plsc.parallel_loop(0, _L * groups_per_j, step=1, unroll=4)
        def grp(g2):
            j = lax.shift_right_logical(g2, 3)
            g = lax.bitwise_and(g2, 7)
            col = g * _LANES
            r16 = rk[j, pl.ds(col, _LANES)]
            s16 = st[j, pl.ds(col, _LANES)]
            cidx = r16 * (5 * _D) + s16 * _D
            for cc in range(_CH):
                vals = plsc.load_gather(ctab_v, [cidx + (c0 + cc)])
                obuf[cc * _L + j, pl.ds(col, _LANES)] = vals

        return [
            pltpu.async_copy(obuf.at[pl.ds(cc * _L, _L), :],
                             out_hbm.at[c0 + cc, :, pl.ds(i0, _ICHUNK)],
                             sem)
            for cc in range(_CH)
        ]

    in_bufs = ((rk0, st0, sin0), (rk1, st1, sin1))
    in_handles = {0: start_in(0, *in_bufs[0])}
    pend = {"a": None, "b": None}

    for kc in range(nchunks):
        rk, st, _ = in_bufs[kc % 2]
        for h in in_handles.pop(kc):
            h.wait()
        if kc + 1 < nchunks:
            in_handles[kc + 1] = start_in(kc + 1, *in_bufs[(kc + 1) % 2])
        for key, c0, obuf, sem in (("a", 0, oa, sa), ("b", _CH, ob, sb)):
            if pend[key] is not None:
                for h in pend[key]:
                    h.wait()
            pend[key] = half(kc, c0, obuf, sem, rk, st)

    for key in ("a", "b"):
        for h in pend[key]:
            h.wait()


@jax.jit
def kernel(ranks, suits, rank_table, suit_table):
    B, L = ranks.shape
    per_w = B // _NUM_WORKERS
    assert per_w * _NUM_WORKERS == B and per_w % _ICHUNK == 0 and L == _L

    mesh = plsc.VectorSubcoreMesh(core_axis_name="c", subcore_axis_name="s")
    res = pl.kernel(
        functools.partial(_card_embed_body, n_i=B),
        out_type=jax.ShapeDtypeStruct((_D, _L, B), jnp.float32),
        mesh=mesh,
        compiler_params=pltpu.CompilerParams(needs_layout_passes=False),
        scratch_types=[
            pltpu.VMEM((14 * 8,), jnp.float32),
            pltpu.VMEM((5 * 4,), jnp.float32),
            pltpu.VMEM((70 * _D,), jnp.float32),
            pltpu.VMEM((_L, _ICHUNK), jnp.int32),
            pltpu.VMEM((_L, _ICHUNK), jnp.int32),
            pltpu.VMEM((_L, _ICHUNK), jnp.int32),
            pltpu.VMEM((_L, _ICHUNK), jnp.int32),
            pltpu.VMEM((_CH * _L, _ICHUNK), jnp.float32),
            pltpu.VMEM((_CH * _L, _ICHUNK), jnp.float32),
            pltpu.SemaphoreType.DMA,
            pltpu.SemaphoreType.DMA,
            pltpu.SemaphoreType.DMA,
            pltpu.SemaphoreType.DMA,
        ],
    )(ranks.T, suits.T, rank_table.reshape(14 * 8), suit_table.reshape(5 * 4))
    return res.transpose(2, 1, 0)
